# Initial kernel scaffold; baseline (speedup 1.0000x reference)
import jax
import jax.numpy as jnp
from jax.experimental import pallas as pl


def kernel(x, edge_index, batch, ctrl, W_pre, b_pre, gin_params, Wl1, bl1, Wl2, bl2, Wl3, bl3, Wl4, bl4, bn_g, bn_b, bn_rm, bn_rv):
    raise NotImplementedError("write your pallas kernel here")


# speedup vs baseline: 5.2299x; 5.2299x over previous
import jax
import jax.numpy as jnp
from jax import lax
from jax.experimental import pallas as pl
from jax.experimental.pallas import tpu as pltpu
from jax.experimental.pallas import tpu_sc as plsc

_N = 10000
_E = 320000
_H = 128
_CHUNK = 128
_NCHUNKS = _E // _CHUNK
_NTILES = 32
_NSUB = 16
_ROWS_PER_TILE = 632
_NPAD = _NSUB * _ROWS_PER_TILE
_EPS = 1e-5

_mesh = plsc.VectorSubcoreMesh(core_axis_name="c", subcore_axis_name="s")


def _seg_sum_body(h_hbm, src_hbm, dst_hbm, zeros_hbm, out_hbm,
                  shared, idx_s, idx_d, rows, sem):
    c = lax.axis_index("c")
    s = lax.axis_index("s")
    wid = s * 2 + c
    pltpu.sync_copy(zeros_hbm,
                    shared.at[pl.ds(s * _ROWS_PER_TILE, _ROWS_PER_TILE)])
    plsc.subcore_barrier()
    nmine = (_NCHUNKS - wid + _NTILES - 1) // _NTILES

    def body(k, carry):
        j = wid + k * _NTILES
        pltpu.sync_copy(src_hbm.at[j], idx_s)
        pltpu.sync_copy(dst_hbm.at[j], idx_d)
        pltpu.async_copy(h_hbm.at[idx_s], rows, sem).wait()
        pltpu.sync_copy(rows, shared.at[idx_d], add=True)
        return carry

    lax.fori_loop(0, nmine, body, 0)
    plsc.subcore_barrier()
    pltpu.sync_copy(shared.at[pl.ds(s * _ROWS_PER_TILE, _ROWS_PER_TILE)],
                    out_hbm.at[c, pl.ds(s * _ROWS_PER_TILE, _ROWS_PER_TILE)])


_seg_sum = pl.kernel(
    _seg_sum_body,
    out_type=jax.ShapeDtypeStruct((2, _NPAD, _H), jnp.float32),
    mesh=_mesh,
    scratch_types=[
        pltpu.VMEM_SHARED((_NPAD, _H), jnp.float32),
        pltpu.VMEM((_CHUNK,), jnp.int32),
        pltpu.VMEM((_CHUNK,), jnp.int32),
        pltpu.VMEM((_CHUNK, _H), jnp.float32),
        pltpu.SemaphoreType.DMA,
    ],
)


def _dirichlet_body(h_hbm, src_hbm, dst_hbm, out_hbm,
                    idx_s, idx_d, rows_s, rows_d, acc_v, sem_s, sem_d):
    c = lax.axis_index("c")
    s = lax.axis_index("s")
    wid = s * 2 + c
    nmine = (_NCHUNKS - wid + _NTILES - 1) // _NTILES

    def chunk_body(k, acc):
        j = wid + k * _NTILES
        pltpu.sync_copy(src_hbm.at[j], idx_s)
        pltpu.sync_copy(dst_hbm.at[j], idx_d)
        cp_s = pltpu.async_copy(h_hbm.at[idx_s], rows_s, sem_s)
        cp_d = pltpu.async_copy(h_hbm.at[idx_d], rows_d, sem_d)
        cp_s.wait()
        cp_d.wait()

        def row_body(i, a):
            for t in range(_H // 16):
                va = rows_s[i, pl.ds(t * 16, 16)]
                vb = rows_d[i, pl.ds(t * 16, 16)]
                dv = va - vb
                a = a + dv * dv
            return a

        return lax.fori_loop(0, _CHUNK, row_body, acc)

    acc = lax.fori_loop(0, nmine, chunk_body, jnp.zeros((16,), jnp.float32))
    acc_v[...] = acc
    pltpu.sync_copy(acc_v, out_hbm.at[pl.ds(wid * 16, 16)])


_dirichlet = pl.kernel(
    _dirichlet_body,
    out_type=jax.ShapeDtypeStruct((_NTILES * 16,), jnp.float32),
    mesh=_mesh,
    scratch_types=[
        pltpu.VMEM((_CHUNK,), jnp.int32),
        pltpu.VMEM((_CHUNK,), jnp.int32),
        pltpu.VMEM((_CHUNK, _H), jnp.float32),
        pltpu.VMEM((_CHUNK, _H), jnp.float32),
        pltpu.VMEM((16,), jnp.float32),
        pltpu.SemaphoreType.DMA,
        pltpu.SemaphoreType.DMA,
    ],
)


_BLK = 1000


def _bn_apply(y, g, bt, rm, rv):
    scale = g / jnp.sqrt(rv + _EPS)
    return y * scale + (bt - rm * scale)


def _pre_body(x_ref, w_ref, b_ref, o_ref):
    y = jnp.dot(x_ref[...], w_ref[...], preferred_element_type=jnp.float32)
    o_ref[...] = jnp.maximum(y + b_ref[...], 0.0)


def _pre_call(x, w, b2d):
    return pl.pallas_call(
        _pre_body,
        grid=(_N // _BLK,),
        in_specs=[
            pl.BlockSpec((_BLK, _H), lambda i: (i, 0)),
            pl.BlockSpec((_H, _H), lambda i: (0, 0)),
            pl.BlockSpec((1, _H), lambda i: (0, 0)),
        ],
        out_specs=pl.BlockSpec((_BLK, _H), lambda i: (i, 0)),
        out_shape=jax.ShapeDtypeStruct((_N, _H), jnp.float32),
    )(x, w, b2d)


def _gin_body(h_ref, agg_ref, w1_ref, f1_ref, w2_ref, f2_ref, o_ref):
    h = h_ref[...]
    m = h + agg_ref[0] + agg_ref[1]
    y = jnp.dot(m, w1_ref[...], preferred_element_type=jnp.float32)
    y = _bn_apply(y + f1_ref[0:1, :], f1_ref[1:2, :], f1_ref[2:3, :],
                  f1_ref[3:4, :], f1_ref[4:5, :])
    y = jnp.maximum(y, 0.0)
    y = jnp.dot(y, w2_ref[...], preferred_element_type=jnp.float32)
    y = _bn_apply(y + f2_ref[0:1, :], f2_ref[1:2, :], f2_ref[2:3, :],
                  f2_ref[3:4, :], f2_ref[4:5, :])
    o_ref[...] = h + jnp.maximum(y, 0.0)


def _gin_call(h, agg, p):
    f1 = jnp.stack([p["b1"], p["g1"], p["bt1"], p["rm1"], p["rv1"]])
    f2 = jnp.stack([p["b2"], p["g2"], p["bt2"], p["rm2"], p["rv2"]])
    return pl.pallas_call(
        _gin_body,
        grid=(_N // _BLK,),
        in_specs=[
            pl.BlockSpec((_BLK, _H), lambda i: (i, 0)),
            pl.BlockSpec((2, _BLK, _H), lambda i: (0, i, 0)),
            pl.BlockSpec((_H, _H), lambda i: (0, 0)),
            pl.BlockSpec((5, _H), lambda i: (0, 0)),
            pl.BlockSpec((_H, _H), lambda i: (0, 0)),
            pl.BlockSpec((5, _H), lambda i: (0, 0)),
        ],
        out_specs=pl.BlockSpec((_BLK, _H), lambda i: (i, 0)),
        out_shape=jax.ShapeDtypeStruct((_N, _H), jnp.float32),
    )(h, agg, p["W1"], f1, p["W2"], f2)


def _head_body(ctrl_ref, h_ref, parts_ref, w3_ref, b3_ref, bn_ref,
               w4_ref, b4_ref, o_ref, de_ref):
    i = ctrl_ref[0]
    hh = h_ref[pl.ds(i, 1), :]
    for t in range(3):
        y = jnp.dot(hh, w3_ref[t], preferred_element_type=jnp.float32)
        y = _bn_apply(y + b3_ref[t:t + 1, :], bn_ref[0:1, :], bn_ref[1:2, :],
                      bn_ref[2:3, :], bn_ref[3:4, :])
        hh = jnp.maximum(y, 0.0)
    o = jnp.dot(hh, w4_ref[...], preferred_element_type=jnp.float32)
    o_ref[...] = o[:, 0:1] + b4_ref[...]
    de_ref[...] = (0.5 * jnp.sum(parts_ref[...]) / _N).reshape(1, 1)


def _head_call(ctrl, h, parts, Wl1, bl1, Wl2, bl2, Wl3, bl3, Wl4, bl4,
               bn_g, bn_b, bn_rm, bn_rv):
    w3 = jnp.stack([Wl1, Wl2, Wl3])
    b3 = jnp.stack([bl1, bl2, bl3])
    bn = jnp.stack([bn_g, bn_b, bn_rm, bn_rv])
    w4 = jnp.zeros((_H, _H), jnp.float32).at[:, 0].set(Wl4[:, 0])
    return pl.pallas_call(
        _head_body,
        in_specs=[
            pl.BlockSpec(memory_space=pltpu.SMEM),
            pl.BlockSpec((_N, _H), lambda: (0, 0)),
            pl.BlockSpec((4, _H), lambda: (0, 0)),
            pl.BlockSpec((3, _H, _H), lambda: (0, 0, 0)),
            pl.BlockSpec((3, _H), lambda: (0, 0)),
            pl.BlockSpec((4, _H), lambda: (0, 0)),
            pl.BlockSpec((_H, _H), lambda: (0, 0)),
            pl.BlockSpec((1, 1), lambda: (0, 0)),
        ],
        out_specs=[
            pl.BlockSpec((1, 1), lambda: (0, 0)),
            pl.BlockSpec((1, 1), lambda: (0, 0)),
        ],
        out_shape=[
            jax.ShapeDtypeStruct((1, 1), jnp.float32),
            jax.ShapeDtypeStruct((1, 1), jnp.float32),
        ],
    )(ctrl, h, parts, w3, b3, bn, w4, bl4.reshape(1, 1))


def kernel(x, edge_index, batch, ctrl, W_pre, b_pre, gin_params,
           Wl1, bl1, Wl2, bl2, Wl3, bl3, Wl4, bl4,
           bn_g, bn_b, bn_rm, bn_rv):
    src2 = edge_index[0].reshape(_NCHUNKS, _CHUNK)
    dst2 = edge_index[1].reshape(_NCHUNKS, _CHUNK)
    zeros = jnp.zeros((_ROWS_PER_TILE, _H), jnp.float32)

    h = _pre_call(x, W_pre, b_pre.reshape(1, _H))
    for p in gin_params:
        agg = _seg_sum(h, src2, dst2, zeros)
        h = _gin_call(h, agg, p)

    parts = _dirichlet(h, src2, dst2)
    o, de = _head_call(ctrl, h, parts.reshape(4, _H),
                       Wl1, bl1, Wl2, bl2, Wl3, bl3, Wl4, bl4,
                       bn_g, bn_b, bn_rm, bn_rv)
    return (o, o, de[0, 0])
